# baseline (speedup 1.0000x reference)
import functools

import jax
import jax.numpy as jnp
from jax import lax
from jax.experimental import pallas as pl
from jax.experimental.pallas import tpu as pltpu
from jax.experimental.pallas import tpu_sc as plsc

PAD = 56
GB = 2
WB = 8


@functools.cache
def _build(batch: int, hist: int, vocab: int, d: int):
  info = plsc.get_sparse_core_info()
  nc, ns = info.num_cores, info.num_subcores
  nw = nc * ns
  per_w = batch // nw
  steps = per_w // WB
  gpg = WB // GB
  assert batch == nw * per_w and per_w % WB == 0 and steps % 2 == 0

  mesh = plsc.VectorSubcoreMesh(core_axis_name="c", subcore_axis_name="s")

  def body(idx_hbm, table_hbm, out_hbm, idx_v, bufs, sems):
    wid = lax.axis_index("s") * nc + lax.axis_index("c")
    obase = wid * per_w
    grows = per_w // GB

    pltpu.sync_copy(idx_hbm.at[pl.ds(wid * grows, grows)], idx_v)

    def fill(i, b):
      for m in range(gpg):
        pltpu.async_copy(
            table_hbm.at[idx_v.at[gpg * i + m, pl.ds(0, GB * PAD)]],
            bufs[b].at[pl.ds(m * GB * PAD, GB * PAD)], sems[b])

    def drain(b):
      for m in range(gpg):
        pltpu.make_async_copy(
            table_hbm.at[idx_v.at[0, pl.ds(0, GB * PAD)]],
            bufs[b].at[pl.ds(m * GB * PAD, GB * PAD)], sems[b]).wait()

    fill(0, 0)

    def step(i2, carry):
      for h in range(2):
        i = 2 * i2 + h
        drain(h)

        @pl.when(i < steps - 1)
        def _(i=i, h=h):
          fill(i + 1, 1 - h)

        pltpu.sync_copy(
            bufs[h].reshape(WB, PAD, d).at[:, pl.ds(0, hist), :],
            out_hbm.at[pl.ds(obase + WB * i, WB)])

      return carry

    lax.fori_loop(0, steps // 2, step, 0)

  return pl.kernel(
      body,
      out_type=jax.ShapeDtypeStruct((batch, hist, d), jnp.float32),
      mesh=mesh,
      scratch_types=[
          pltpu.VMEM((batch // nw // GB, 128), jnp.int32),
          [pltpu.VMEM((WB * PAD, d), jnp.float32) for _ in range(2)],
          [pltpu.SemaphoreType.DMA for _ in range(2)],
      ],
  )


@jax.jit
def kernel(x, table):
  b, h = x.shape
  vocab, d = table.shape
  fill = jnp.arange(b * (PAD - h), dtype=jnp.int32).reshape(b, PAD - h) % vocab
  xp = jnp.concatenate([x.astype(jnp.int32), fill], axis=1)
  idx = xp.reshape(b // GB, GB * PAD)
  idx = jnp.pad(idx, ((0, 0), (0, 128 - GB * PAD)))
  return _build(b, h, vocab, d)(idx, table)


# speedup vs baseline: 1.9087x; 1.9087x over previous
import functools

import jax
import jax.numpy as jnp
from jax import lax
from jax.experimental import pallas as pl
from jax.experimental.pallas import tpu as pltpu
from jax.experimental.pallas import tpu_sc as plsc


@functools.cache
def _build(batch: int, hist: int, vocab: int, d: int):
  info = plsc.get_sparse_core_info()
  nc, ns = info.num_cores, info.num_subcores
  nw = nc * ns
  bpw = batch // nw
  nbuf = 5
  steps = hist // nbuf
  assert batch == nw * bpw and hist == steps * nbuf

  mesh = plsc.VectorSubcoreMesh(core_axis_name="c", subcore_axis_name="s")

  def body(idx_hbm, table_hbm, out_hbm, idx_v, bufs, sems):
    wid = lax.axis_index("s") * nc + lax.axis_index("c")
    b0 = wid * bpw

    pltpu.sync_copy(idx_hbm.at[:, pl.ds(b0, bpw)], idx_v)

    def gather(h, b):
      pltpu.async_copy(table_hbm.at[idx_v.at[h]], bufs[b], sems[b])

    def gwait(b):
      pltpu.make_async_copy(
          table_hbm.at[idx_v.at[0]], bufs[b], sems[b]).wait()

    for b in range(nbuf):
      gather(b, b)

    def step(i, carry):
      h0 = nbuf * i
      for b in range(nbuf):
        gwait(b)
        pltpu.sync_copy(bufs[b], out_hbm.at[h0 + b, pl.ds(b0, bpw)])

        @pl.when(i < steps - 1)
        def _(b=b):
          gather(h0 + nbuf + b, b)

      return carry

    lax.fori_loop(0, steps, step, 0)

  return pl.kernel(
      body,
      out_type=jax.ShapeDtypeStruct((hist, batch, d), jnp.float32),
      mesh=mesh,
      scratch_types=[
          pltpu.VMEM((hist, bpw), jnp.int32),
          [pltpu.VMEM((bpw, d), jnp.float32) for _ in range(nbuf)],
          [pltpu.SemaphoreType.DMA for _ in range(nbuf)],
      ],
  )


@jax.jit
def kernel(x, table):
  b, h = x.shape
  vocab, d = table.shape
  out_t = _build(b, h, vocab, d)(x.T.astype(jnp.int32), table)
  return out_t.transpose(1, 0, 2)
